# initial kernel scaffold (unmeasured)
import jax
import jax.numpy as jnp
from jax import lax
from jax.experimental import pallas as pl
from jax.experimental.pallas import tpu as pltpu

N_DEV = 16
H_CW = 8
H_CCW = 7

_RING = [0, 4, 8, 12, 13, 9, 5, 1, 2, 6, 10, 14, 15, 11, 7, 3]
_INV = [0] * N_DEV
for _p, _l in enumerate(_RING):
    _INV[_l] = _p


def kernel(x, w_mat):
    m_per, k = x.shape
    n_out = w_mat.shape[1]
    m_total = N_DEV * m_per

    ring = jnp.asarray(_RING, dtype=jnp.int32)
    inv = jnp.asarray(_INV, dtype=jnp.int32)

    my = lax.axis_index("i").astype(jnp.int32)
    pos = jnp.take(inv, my)
    nbrs = jnp.stack(
        [
            jnp.take(ring, (pos - 1) % N_DEV),
            jnp.take(ring, (pos + 1) % N_DEV),
        ]
    )
    orig_cw = jnp.take(ring, (pos - 1 - jnp.arange(H_CW, dtype=jnp.int32)) % N_DEV)
    orig_ccw = jnp.take(ring, (pos + 1 + jnp.arange(H_CCW, dtype=jnp.int32)) % N_DEV)

    def body(
        x_ref, w_ref, nbrs_ref, ocw_ref, occw_ref, out_ref,
        bcw, bccw, scw_sem, rcw_sem, sccw_sem, rccw_sem, cred_cw, cred_ccw,
    ):
        left = nbrs_ref[0]
        right = nbrs_ref[1]

        barrier_sem = pltpu.get_barrier_semaphore()
        for nbr in (left, right):
            pl.semaphore_signal(
                barrier_sem, inc=1,
                device_id=(nbr,), device_id_type=pl.DeviceIdType.MESH,
            )
        pl.semaphore_wait(barrier_sem, 2)

        def rdma(h, cw):
            buf, s_sem, r_sem = (bcw, scw_sem, rcw_sem) if cw else (bccw, sccw_sem, rccw_sem)
            src = x_ref if h == 0 else buf.at[(h - 1) % 2]
            return pltpu.make_async_remote_copy(
                src_ref=src,
                dst_ref=buf.at[h % 2],
                send_sem=s_sem.at[h % 2],
                recv_sem=r_sem.at[h % 2],
                device_id=(right if cw else left,),
                device_id_type=pl.DeviceIdType.MESH,
            )

        def gemm(chunk, origin):
            y = jnp.dot(chunk, w_ref[...], preferred_element_type=jnp.float32)
            out_ref[pl.ds(origin * m_per, m_per), :] = jnp.maximum(y, 0.0)

        for h in range(H_CW):
            if h >= 2:
                pl.semaphore_wait(cred_cw, 1)
            cw = rdma(h, cw=True)
            cw.start()
            if h < H_CCW:
                if h >= 2:
                    pl.semaphore_wait(cred_ccw, 1)
                ccw = rdma(h, cw=False)
                ccw.start()

            if h == 0:
                gemm(x_ref[...], my)

            cw.wait()
            gemm(bcw[h % 2], ocw_ref[h])
            if h < H_CCW:
                ccw.wait()
                gemm(bccw[h % 2], occw_ref[h])

            if 1 <= h <= H_CW - 2:
                pl.semaphore_signal(
                    cred_cw, inc=1,
                    device_id=(left,), device_id_type=pl.DeviceIdType.MESH,
                )
            if 1 <= h <= H_CCW - 2:
                pl.semaphore_signal(
                    cred_ccw, inc=1,
                    device_id=(right,), device_id_type=pl.DeviceIdType.MESH,
                )

    out_shape = jax.ShapeDtypeStruct((m_total, n_out), jnp.float32)
    return pl.pallas_call(
        body,
        out_shape=out_shape,
        in_specs=[
            pl.BlockSpec(memory_space=pltpu.VMEM),
            pl.BlockSpec(memory_space=pltpu.VMEM),
            pl.BlockSpec(memory_space=pltpu.SMEM),
            pl.BlockSpec(memory_space=pltpu.SMEM),
            pl.BlockSpec(memory_space=pltpu.SMEM),
        ],
        out_specs=pl.BlockSpec(memory_space=pltpu.VMEM),
        scratch_shapes=[
            pltpu.VMEM((2, m_per, k), jnp.float32),
            pltpu.VMEM((2, m_per, k), jnp.float32),
            pltpu.SemaphoreType.DMA((2,)),
            pltpu.SemaphoreType.DMA((2,)),
            pltpu.SemaphoreType.DMA((2,)),
            pltpu.SemaphoreType.DMA((2,)),
            pltpu.SemaphoreType.REGULAR,
            pltpu.SemaphoreType.REGULAR,
        ],
        compiler_params=pltpu.CompilerParams(collective_id=0),
    )(x, w_mat, nbrs, orig_cw, orig_ccw)


# baseline (device time: 410538 ns/iter reference)
import jax
import jax.numpy as jnp
from jax import lax
from jax.experimental import pallas as pl
from jax.experimental.pallas import tpu as pltpu

N_DEV = 16
H_CW = 8
H_CCW = 7

_RING = [0, 4, 8, 12, 13, 9, 5, 1, 2, 6, 10, 14, 15, 11, 7, 3]
_INV = [0] * N_DEV
for _p, _l in enumerate(_RING):
    _INV[_l] = _p


def kernel(x, w_mat):
    m_per, k = x.shape
    n_out = w_mat.shape[1]
    m_total = N_DEV * m_per

    ring = jnp.asarray(_RING, dtype=jnp.int32)
    inv = jnp.asarray(_INV, dtype=jnp.int32)

    my = lax.axis_index("i").astype(jnp.int32)
    pos = jnp.take(inv, my)
    nbrs = jnp.stack(
        [
            jnp.take(ring, (pos - 1) % N_DEV),
            jnp.take(ring, (pos + 1) % N_DEV),
        ]
    )
    orig_cw = jnp.take(ring, (pos - 1 - jnp.arange(H_CW, dtype=jnp.int32)) % N_DEV)
    orig_ccw = jnp.take(ring, (pos + 1 + jnp.arange(H_CCW, dtype=jnp.int32)) % N_DEV)

    def body(
        x_ref, w_ref, nbrs_ref, ocw_ref, occw_ref, out_ref,
        bcw, bccw, scw_sem, rcw_sem, sccw_sem, rccw_sem, cred_cw, cred_ccw,
    ):
        left = nbrs_ref[0]
        right = nbrs_ref[1]

        barrier_sem = pltpu.get_barrier_semaphore()
        for nbr in (left, right):
            pl.semaphore_signal(
                barrier_sem, inc=1,
                device_id=(nbr,), device_id_type=pl.DeviceIdType.MESH,
            )
        pl.semaphore_wait(barrier_sem, 2)

        def rdma(h, cw):
            buf, s_sem, r_sem = (bcw, scw_sem, rcw_sem) if cw else (bccw, sccw_sem, rccw_sem)
            src = x_ref if h == 0 else buf.at[(h - 1) % 2]
            return pltpu.make_async_remote_copy(
                src_ref=src,
                dst_ref=buf.at[h % 2],
                send_sem=s_sem.at[h % 2],
                recv_sem=r_sem.at[h % 2],
                device_id=(right if cw else left,),
                device_id_type=pl.DeviceIdType.MESH,
            )

        def gemm(chunk, origin):
            y = jnp.dot(chunk, w_ref[...], preferred_element_type=jnp.float32)
            out_ref[pl.ds(origin * m_per, m_per), :] = jnp.maximum(y, 0.0)

        for h in range(H_CW):
            if h >= 2:
                pl.semaphore_wait(cred_cw, 1)
            cw = rdma(h, cw=True)
            cw.start()
            if h < H_CCW:
                if h >= 2:
                    pl.semaphore_wait(cred_ccw, 1)
                ccw = rdma(h, cw=False)
                ccw.start()

            if h == 0:
                gemm(x_ref[...], lax.axis_index("i").astype(jnp.int32))

            cw.wait()
            gemm(bcw[h % 2], ocw_ref[h])
            if h < H_CCW:
                ccw.wait()
                gemm(bccw[h % 2], occw_ref[h])

            if 1 <= h <= H_CW - 2:
                pl.semaphore_signal(
                    cred_cw, inc=1,
                    device_id=(left,), device_id_type=pl.DeviceIdType.MESH,
                )
            if 1 <= h <= H_CCW - 2:
                pl.semaphore_signal(
                    cred_ccw, inc=1,
                    device_id=(right,), device_id_type=pl.DeviceIdType.MESH,
                )

    out_shape = jax.ShapeDtypeStruct((m_total, n_out), jnp.float32)
    return pl.pallas_call(
        body,
        out_shape=out_shape,
        in_specs=[
            pl.BlockSpec(memory_space=pltpu.VMEM),
            pl.BlockSpec(memory_space=pltpu.VMEM),
            pl.BlockSpec(memory_space=pltpu.SMEM),
            pl.BlockSpec(memory_space=pltpu.SMEM),
            pl.BlockSpec(memory_space=pltpu.SMEM),
        ],
        out_specs=pl.BlockSpec(memory_space=pltpu.VMEM),
        scratch_shapes=[
            pltpu.VMEM((2, m_per, k), jnp.float32),
            pltpu.VMEM((2, m_per, k), jnp.float32),
            pltpu.SemaphoreType.DMA((2,)),
            pltpu.SemaphoreType.DMA((2,)),
            pltpu.SemaphoreType.DMA((2,)),
            pltpu.SemaphoreType.DMA((2,)),
            pltpu.SemaphoreType.REGULAR,
            pltpu.SemaphoreType.REGULAR,
        ],
        compiler_params=pltpu.CompilerParams(collective_id=0),
    )(x, w_mat, nbrs, orig_cw, orig_ccw)


# device time: 370350 ns/iter; 1.1085x vs baseline; 1.1085x over previous
import jax
import jax.numpy as jnp
from jax import lax
from jax.experimental import pallas as pl
from jax.experimental.pallas import tpu as pltpu

N_DEV = 16
H = 8

_RING = [0, 4, 8, 12, 13, 9, 5, 1, 2, 6, 10, 14, 15, 11, 7, 3]
_INV = [0] * N_DEV
for _p, _l in enumerate(_RING):
    _INV[_l] = _p


def kernel(x, w_mat):
    m_per, k = x.shape
    n_out = w_mat.shape[1]
    m_half = m_per // 2
    m_total = N_DEV * m_per

    ring = jnp.asarray(_RING, dtype=jnp.int32)
    inv = jnp.asarray(_INV, dtype=jnp.int32)

    my = lax.axis_index("i").astype(jnp.int32)
    pos = jnp.take(inv, my)
    nbrs = jnp.stack(
        [
            jnp.take(ring, (pos - 1) % N_DEV),
            jnp.take(ring, (pos + 1) % N_DEV),
        ]
    )
    orig_cw = jnp.take(ring, (pos - 1 - jnp.arange(H, dtype=jnp.int32)) % N_DEV)
    orig_ccw = jnp.take(ring, (pos + 1 + jnp.arange(H, dtype=jnp.int32)) % N_DEV)

    def body(
        x_ref, w_ref, nbrs_ref, ocw_ref, occw_ref, out_ref,
        bcw, bccw, scw_sem, rcw_sem, sccw_sem, rccw_sem, cred_cw, cred_ccw,
    ):
        left = nbrs_ref[0]
        right = nbrs_ref[1]

        barrier_sem = pltpu.get_barrier_semaphore()
        for nbr in (left, right):
            pl.semaphore_signal(
                barrier_sem, inc=1,
                device_id=(nbr,), device_id_type=pl.DeviceIdType.MESH,
            )
        pl.semaphore_wait(barrier_sem, 2)

        def desc(h, cw):
            buf, s_sem, r_sem = (
                (bcw, scw_sem, rcw_sem) if cw else (bccw, sccw_sem, rccw_sem)
            )
            rows = pl.ds(0, m_half) if cw else pl.ds(m_half, m_half)
            if h == 0:
                src = x_ref
                dst = buf.at[0]
            elif h < H - 1:
                src = buf.at[(h - 1) % 2]
                dst = buf.at[h % 2]
            else:
                src = buf.at[(h - 1) % 2, rows]
                dst = buf.at[h % 2, rows]
            return pltpu.make_async_remote_copy(
                src_ref=src,
                dst_ref=dst,
                send_sem=s_sem.at[h % 2],
                recv_sem=r_sem.at[h % 2],
                device_id=(right if cw else left,),
                device_id_type=pl.DeviceIdType.MESH,
            )

        def gemm(chunk, origin, row_off=0, rows=m_per):
            y = jnp.dot(chunk, w_ref[...], preferred_element_type=jnp.float32)
            out_ref[pl.ds(origin * m_per + row_off, rows), :] = jnp.maximum(y, 0.0)

        desc(0, cw=True).start()
        desc(0, cw=False).start()
        gemm(x_ref[...], lax.axis_index("i").astype(jnp.int32))

        for h in range(H):
            for cw in (True, False):
                cred = cred_cw if cw else cred_ccw
                desc(h, cw).wait_recv()
                desc(h, cw).wait_send()
                if 1 <= h <= H - 2:
                    pl.semaphore_signal(
                        cred, inc=1,
                        device_id=(left if cw else right,),
                        device_id_type=pl.DeviceIdType.MESH,
                    )
                if h + 1 < H:
                    if h + 1 >= 2:
                        pl.semaphore_wait(cred, 1)
                    desc(h + 1, cw).start()
            if h < H - 1:
                gemm(bcw[h % 2], ocw_ref[h])
                gemm(bccw[h % 2], occw_ref[h])
            else:
                gemm(bcw[h % 2, pl.ds(0, m_half)], ocw_ref[h], 0, m_half)
                gemm(bccw[h % 2, pl.ds(m_half, m_half)], occw_ref[h], m_half, m_half)

    out_shape = jax.ShapeDtypeStruct((m_total, n_out), jnp.float32)
    return pl.pallas_call(
        body,
        out_shape=out_shape,
        in_specs=[
            pl.BlockSpec(memory_space=pltpu.VMEM),
            pl.BlockSpec(memory_space=pltpu.VMEM),
            pl.BlockSpec(memory_space=pltpu.SMEM),
            pl.BlockSpec(memory_space=pltpu.SMEM),
            pl.BlockSpec(memory_space=pltpu.SMEM),
        ],
        out_specs=pl.BlockSpec(memory_space=pltpu.VMEM),
        scratch_shapes=[
            pltpu.VMEM((2, m_per, k), jnp.float32),
            pltpu.VMEM((2, m_per, k), jnp.float32),
            pltpu.SemaphoreType.DMA((2,)),
            pltpu.SemaphoreType.DMA((2,)),
            pltpu.SemaphoreType.DMA((2,)),
            pltpu.SemaphoreType.DMA((2,)),
            pltpu.SemaphoreType.REGULAR,
            pltpu.SemaphoreType.REGULAR,
        ],
        compiler_params=pltpu.CompilerParams(collective_id=0),
    )(x, w_mat, nbrs, orig_cw, orig_ccw)


# device time: 366628 ns/iter; 1.1198x vs baseline; 1.0102x over previous
import jax
import jax.numpy as jnp
from jax import lax
from jax.experimental import pallas as pl
from jax.experimental.pallas import tpu as pltpu

N_DEV = 16
H = 8
S = 3

_RING = [0, 4, 8, 12, 13, 9, 5, 1, 2, 6, 10, 14, 15, 11, 7, 3]
_INV = [0] * N_DEV
for _p, _l in enumerate(_RING):
    _INV[_l] = _p


def kernel(x, w_mat):
    m_per, k = x.shape
    n_out = w_mat.shape[1]
    m_half = m_per // 2
    m_total = N_DEV * m_per

    ring = jnp.asarray(_RING, dtype=jnp.int32)
    inv = jnp.asarray(_INV, dtype=jnp.int32)

    my = lax.axis_index("i").astype(jnp.int32)
    pos = jnp.take(inv, my)
    nbrs = jnp.stack(
        [
            jnp.take(ring, (pos - 1) % N_DEV),
            jnp.take(ring, (pos + 1) % N_DEV),
        ]
    )
    orig_cw = jnp.take(ring, (pos - 1 - jnp.arange(H, dtype=jnp.int32)) % N_DEV)
    orig_ccw = jnp.take(ring, (pos + 1 + jnp.arange(H, dtype=jnp.int32)) % N_DEV)

    def body(
        x_ref, w_ref, nbrs_ref, ocw_ref, occw_ref, out_ref,
        bcw, bccw, scw_sem, rcw_sem, sccw_sem, rccw_sem, cred_cw, cred_ccw,
    ):
        left = nbrs_ref[0]
        right = nbrs_ref[1]

        barrier_sem = pltpu.get_barrier_semaphore()
        for nbr in (left, right):
            pl.semaphore_signal(
                barrier_sem, inc=1,
                device_id=(nbr,), device_id_type=pl.DeviceIdType.MESH,
            )
        pl.semaphore_wait(barrier_sem, 2)

        def desc(h, cw):
            buf, s_sem, r_sem = (
                (bcw, scw_sem, rcw_sem) if cw else (bccw, sccw_sem, rccw_sem)
            )
            rows = pl.ds(0, m_half) if cw else pl.ds(m_half, m_half)
            if h == 0:
                src = x_ref
                dst = buf.at[0]
            elif h < H - 1:
                src = buf.at[(h - 1) % S]
                dst = buf.at[h % S]
            else:
                src = buf.at[(h - 1) % S, rows]
                dst = buf.at[h % S, rows]
            return pltpu.make_async_remote_copy(
                src_ref=src,
                dst_ref=dst,
                send_sem=s_sem.at[h % S],
                recv_sem=r_sem.at[h % S],
                device_id=(right if cw else left,),
                device_id_type=pl.DeviceIdType.MESH,
            )

        def gemm(chunk, origin, row_off=0, rows=m_per):
            y = jnp.dot(chunk, w_ref[...], preferred_element_type=jnp.float32)
            out_ref[pl.ds(origin * m_per + row_off, rows), :] = jnp.maximum(y, 0.0)

        desc(0, cw=True).start()
        desc(0, cw=False).start()
        gemm(x_ref[...], lax.axis_index("i").astype(jnp.int32))

        for h in range(H):
            for cw in (True, False):
                cred = cred_cw if cw else cred_ccw
                desc(h, cw).wait_recv()
                desc(h, cw).wait_send()
                if 1 <= h <= H - S:
                    pl.semaphore_signal(
                        cred, inc=1,
                        device_id=(left if cw else right,),
                        device_id_type=pl.DeviceIdType.MESH,
                    )
                if h + 1 < H:
                    if h + 1 >= S:
                        pl.semaphore_wait(cred, 1)
                    desc(h + 1, cw).start()
            if h < H - 1:
                gemm(bcw[h % S], ocw_ref[h])
                gemm(bccw[h % S], occw_ref[h])
            else:
                gemm(bcw[h % S, pl.ds(0, m_half)], ocw_ref[h], 0, m_half)
                gemm(bccw[h % S, pl.ds(m_half, m_half)], occw_ref[h], m_half, m_half)

    out_shape = jax.ShapeDtypeStruct((m_total, n_out), jnp.float32)
    return pl.pallas_call(
        body,
        out_shape=out_shape,
        in_specs=[
            pl.BlockSpec(memory_space=pltpu.VMEM),
            pl.BlockSpec(memory_space=pltpu.VMEM),
            pl.BlockSpec(memory_space=pltpu.SMEM),
            pl.BlockSpec(memory_space=pltpu.SMEM),
            pl.BlockSpec(memory_space=pltpu.SMEM),
        ],
        out_specs=pl.BlockSpec(memory_space=pltpu.VMEM),
        scratch_shapes=[
            pltpu.VMEM((S, m_per, k), jnp.float32),
            pltpu.VMEM((S, m_per, k), jnp.float32),
            pltpu.SemaphoreType.DMA((S,)),
            pltpu.SemaphoreType.DMA((S,)),
            pltpu.SemaphoreType.DMA((S,)),
            pltpu.SemaphoreType.DMA((S,)),
            pltpu.SemaphoreType.REGULAR,
            pltpu.SemaphoreType.REGULAR,
        ],
        compiler_params=pltpu.CompilerParams(collective_id=0),
    )(x, w_mat, nbrs, orig_cw, orig_ccw)


# device time: 354977 ns/iter; 1.1565x vs baseline; 1.0328x over previous
import jax
import jax.numpy as jnp
from jax import lax
from jax.experimental import pallas as pl
from jax.experimental.pallas import tpu as pltpu

N_DEV = 16
H = 8
S = 3
P = 2

_RING = [0, 4, 8, 12, 13, 9, 5, 1, 2, 6, 10, 14, 15, 11, 7, 3]
_INV = [0] * N_DEV
for _p, _l in enumerate(_RING):
    _INV[_l] = _p


def kernel(x, w_mat):
    m_per, k = x.shape
    n_out = w_mat.shape[1]
    m_half = m_per // 2
    m_total = N_DEV * m_per

    ring = jnp.asarray(_RING, dtype=jnp.int32)
    inv = jnp.asarray(_INV, dtype=jnp.int32)

    my = lax.axis_index("i").astype(jnp.int32)
    pos = jnp.take(inv, my)
    nbrs = jnp.stack(
        [
            jnp.take(ring, (pos - 1) % N_DEV),
            jnp.take(ring, (pos + 1) % N_DEV),
        ]
    )
    orig_cw = jnp.take(ring, (pos - 1 - jnp.arange(H, dtype=jnp.int32)) % N_DEV)
    orig_ccw = jnp.take(ring, (pos + 1 + jnp.arange(H, dtype=jnp.int32)) % N_DEV)

    def body(
        x_ref, w_ref, nbrs_ref, ocw_ref, occw_ref, out_ref,
        bcw, bccw, scw_sem, rcw_sem, sccw_sem, rccw_sem, cred_cw, cred_ccw,
    ):
        left = nbrs_ref[0]
        right = nbrs_ref[1]

        barrier_sem = pltpu.get_barrier_semaphore()
        for nbr in (left, right):
            pl.semaphore_signal(
                barrier_sem, inc=1,
                device_id=(nbr,), device_id_type=pl.DeviceIdType.MESH,
            )
        pl.semaphore_wait(barrier_sem, 2)

        def desc(h, cw, p):
            buf, s_sem, r_sem = (
                (bcw, scw_sem, rcw_sem) if cw else (bccw, sccw_sem, rccw_sem)
            )
            if h < H - 1:
                rows = pl.ds(p * (m_per // P), m_per // P)
            else:
                off = 0 if cw else m_half
                rows = pl.ds(off + p * (m_half // P), m_half // P)
            src = x_ref.at[rows] if h == 0 else buf.at[(h - 1) % S, rows]
            return pltpu.make_async_remote_copy(
                src_ref=src,
                dst_ref=buf.at[h % S, rows],
                send_sem=s_sem.at[h % S, p],
                recv_sem=r_sem.at[h % S, p],
                device_id=(right if cw else left,),
                device_id_type=pl.DeviceIdType.MESH,
            )

        def gemm(chunk, origin, row_off=0, rows=m_per):
            y = jnp.dot(chunk, w_ref[...], preferred_element_type=jnp.float32)
            out_ref[pl.ds(origin * m_per + row_off, rows), :] = jnp.maximum(y, 0.0)

        for p in range(P):
            desc(0, True, p).start()
            desc(0, False, p).start()
        gemm(x_ref[...], lax.axis_index("i").astype(jnp.int32))

        for h in range(H):
            for cw in (True, False):
                cred = cred_cw if cw else cred_ccw
                for p in range(P):
                    desc(h, cw, p).wait_recv()
                    if h + 1 < H:
                        if p == 0 and h + 1 >= S:
                            pl.semaphore_wait(cred, 1)
                        desc(h + 1, cw, p).start()
                for p in range(P):
                    desc(h, cw, p).wait_send()
                if 1 <= h <= H - S:
                    pl.semaphore_signal(
                        cred, inc=1,
                        device_id=(left if cw else right,),
                        device_id_type=pl.DeviceIdType.MESH,
                    )
            if h < H - 1:
                gemm(bcw[h % S], ocw_ref[h])
                gemm(bccw[h % S], occw_ref[h])
            else:
                gemm(bcw[h % S, pl.ds(0, m_half)], ocw_ref[h], 0, m_half)
                gemm(bccw[h % S, pl.ds(m_half, m_half)], occw_ref[h], m_half, m_half)

    out_shape = jax.ShapeDtypeStruct((m_total, n_out), jnp.float32)
    return pl.pallas_call(
        body,
        out_shape=out_shape,
        in_specs=[
            pl.BlockSpec(memory_space=pltpu.VMEM),
            pl.BlockSpec(memory_space=pltpu.VMEM),
            pl.BlockSpec(memory_space=pltpu.SMEM),
            pl.BlockSpec(memory_space=pltpu.SMEM),
            pl.BlockSpec(memory_space=pltpu.SMEM),
        ],
        out_specs=pl.BlockSpec(memory_space=pltpu.VMEM),
        scratch_shapes=[
            pltpu.VMEM((S, m_per, k), jnp.float32),
            pltpu.VMEM((S, m_per, k), jnp.float32),
            pltpu.SemaphoreType.DMA((S, P)),
            pltpu.SemaphoreType.DMA((S, P)),
            pltpu.SemaphoreType.DMA((S, P)),
            pltpu.SemaphoreType.DMA((S, P)),
            pltpu.SemaphoreType.REGULAR,
            pltpu.SemaphoreType.REGULAR,
        ],
        compiler_params=pltpu.CompilerParams(collective_id=0),
    )(x, w_mat, nbrs, orig_cw, orig_ccw)


# device time: 353771 ns/iter; 1.1605x vs baseline; 1.0034x over previous
import jax
import jax.numpy as jnp
from jax import lax
from jax.experimental import pallas as pl
from jax.experimental.pallas import tpu as pltpu

N_DEV = 16
H = 8
S = 3
P = 2

_RING = [0, 4, 8, 12, 13, 9, 5, 1, 2, 6, 10, 14, 15, 11, 7, 3]
_INV = [0] * N_DEV
for _p, _l in enumerate(_RING):
    _INV[_l] = _p


def kernel(x, w_mat):
    m_per, k = x.shape
    n_out = w_mat.shape[1]
    m_half = m_per // 2
    m_total = N_DEV * m_per

    ring = jnp.asarray(_RING, dtype=jnp.int32)
    inv = jnp.asarray(_INV, dtype=jnp.int32)

    my = lax.axis_index("i").astype(jnp.int32)
    pos = jnp.take(inv, my)
    nbrs = jnp.stack(
        [
            jnp.take(ring, (pos - 1) % N_DEV),
            jnp.take(ring, (pos + 1) % N_DEV),
        ]
    )
    orig_cw = jnp.take(ring, (pos - 1 - jnp.arange(H, dtype=jnp.int32)) % N_DEV)
    orig_ccw = jnp.take(ring, (pos + 1 + jnp.arange(H, dtype=jnp.int32)) % N_DEV)

    def body(
        x_ref, w_ref, nbrs_ref, ocw_ref, occw_ref, out_ref,
        bcw, bccw, scw_sem, rcw_sem, sccw_sem, rccw_sem, cred_cw, cred_ccw,
    ):
        left = nbrs_ref[0]
        right = nbrs_ref[1]

        barrier_sem = pltpu.get_barrier_semaphore()
        for nbr in (left, right):
            pl.semaphore_signal(
                barrier_sem, inc=1,
                device_id=(nbr,), device_id_type=pl.DeviceIdType.MESH,
            )
        pl.semaphore_wait(barrier_sem, 2)

        def desc(h, cw, p):
            buf, s_sem, r_sem = (
                (bcw, scw_sem, rcw_sem) if cw else (bccw, sccw_sem, rccw_sem)
            )
            if h < H - 1:
                rows = pl.ds(p * (m_per // P), m_per // P)
            else:
                off = 0 if cw else m_half
                rows = pl.ds(off + p * (m_half // P), m_half // P)
            src = x_ref.at[rows] if h == 0 else buf.at[(h - 1) % S, rows]
            return pltpu.make_async_remote_copy(
                src_ref=src,
                dst_ref=buf.at[h % S, rows],
                send_sem=s_sem.at[h % S, p],
                recv_sem=r_sem.at[h % S, p],
                device_id=(right if cw else left,),
                device_id_type=pl.DeviceIdType.MESH,
            )

        def gemm(chunk, origin, row_off=0, rows=m_per):
            y = jnp.dot(chunk, w_ref[...], preferred_element_type=jnp.float32)
            out_ref[pl.ds(origin * m_per + row_off, rows), :] = jnp.maximum(y, 0.0)

        for p in range(P):
            desc(0, True, p).start()
            desc(0, False, p).start()
        gemm(x_ref[...], lax.axis_index("i").astype(jnp.int32))

        for h in range(H):
            for cw in (True, False):
                cred = cred_cw if cw else cred_ccw
                for p in range(P):
                    desc(h, cw, p).wait_recv()
                    if h + 1 < H:
                        if p == 0 and h + 1 >= S:
                            pl.semaphore_wait(cred, 1)
                        desc(h + 1, cw, p).start()
                    else:
                        pr = m_half // P
                        off = (0 if cw else m_half) + p * pr
                        buf = bcw if cw else bccw
                        gemm(buf[h % S, pl.ds(off, pr)], ocw_ref[h], off, pr)
                for p in range(P):
                    desc(h, cw, p).wait_send()
                if 1 <= h <= H - S:
                    pl.semaphore_signal(
                        cred, inc=1,
                        device_id=(left if cw else right,),
                        device_id_type=pl.DeviceIdType.MESH,
                    )
            if h < H - 1:
                gemm(bcw[h % S], ocw_ref[h])
                gemm(bccw[h % S], occw_ref[h])

    out_shape = jax.ShapeDtypeStruct((m_total, n_out), jnp.float32)
    return pl.pallas_call(
        body,
        out_shape=out_shape,
        in_specs=[
            pl.BlockSpec(memory_space=pltpu.VMEM),
            pl.BlockSpec(memory_space=pltpu.VMEM),
            pl.BlockSpec(memory_space=pltpu.SMEM),
            pl.BlockSpec(memory_space=pltpu.SMEM),
            pl.BlockSpec(memory_space=pltpu.SMEM),
        ],
        out_specs=pl.BlockSpec(memory_space=pltpu.VMEM),
        scratch_shapes=[
            pltpu.VMEM((S, m_per, k), jnp.float32),
            pltpu.VMEM((S, m_per, k), jnp.float32),
            pltpu.SemaphoreType.DMA((S, P)),
            pltpu.SemaphoreType.DMA((S, P)),
            pltpu.SemaphoreType.DMA((S, P)),
            pltpu.SemaphoreType.DMA((S, P)),
            pltpu.SemaphoreType.REGULAR,
            pltpu.SemaphoreType.REGULAR,
        ],
        compiler_params=pltpu.CompilerParams(collective_id=0),
    )(x, w_mat, nbrs, orig_cw, orig_ccw)
